# baseline (speedup 1.0000x reference)
import jax
import jax.numpy as jnp
import numpy as np
from jax import lax
from jax.experimental import pallas as pl
from jax.experimental.pallas import tpu as pltpu
from jax.experimental.pallas import tpu_sc as plsc

NC, NS, L = 2, 16, 16
E = 256
NR = 100000
RCH = 6256

_BASE = np.array([1, 0, 0, 0, 0, 1, 0, 0, 0, 0, 1, 0, 0, 0, 0, 1], np.float32)
_C1 = np.array([-2, 2, 2, 0, 2, -2, 2, 0, 2, 2, -2, 0, 0, 0, 0, 0], np.float32)
_C2 = np.array([-2, -2, 2, 0, 2, -2, -2, 0, -2, 2, -2, 0, 0, 0, 0, 0], np.float32)
_TCOEF = np.array([0, 0, 0, 1, 0, 0, 0, 1, 0, 0, 0, 1, 0, 0, 0, 0], np.float32)
_A = np.array([2, 1, 0, 0, 1, 1, 2, 0, 1, 0, 1, 0, 0, 0, 0, 0], np.int32)
_B = np.array([2, 2, 2, 0, 2, 1, 3, 0, 3, 1, 1, 0, 0, 0, 0, 0], np.int32)
_C = np.array([3, 0, 1, 0, 0, 3, 0, 0, 0, 2, 2, 0, 0, 0, 0, 0], np.int32)
_D = np.array([3, 3, 3, 0, 3, 3, 1, 0, 2, 3, 2, 0, 0, 0, 0, 0], np.int32)
_TIDX = np.array([0, 0, 0, 0, 0, 0, 0, 1, 0, 0, 0, 2, 0, 0, 0, 0], np.int32)
_PAT3 = (np.arange(16) % 3).astype(np.int32)
_PAT4 = (np.arange(16) % 4).astype(np.int32)
_CI = np.stack([_A, _B, _C, _D, _TIDX, _PAT3, _PAT4,
                np.arange(16, dtype=np.int32)])
_CF = np.stack([_BASE, _C1, _C2, _TCOEF])
_IIDX = np.arange(256, dtype=np.int32).reshape(2, 128)


def _body(idx16, embt, w1t, b1t, w2t, b2t, w3t, b3t,
          embr, w1r, b1r, w2r, b2r, w3r, b3r,
          tb, rb, ci, cf, iidx,
          c2w_o, tbn_o, rbn_o,
          idx_v, ci_v, cf_v, ii_v, e_v, w1_v, w2_v, w3_v,
          part_v, p3_v, hc_v, bias_v, t_v, r_v, bt_v, br_v,
          shA_t, shB_t, sh3_t, shA_r, shB_r, sh3_r,
          sem_in, sem_out):
  cid = lax.axis_index("c")
  sid = lax.axis_index("s")
  z16 = jnp.zeros((L,), jnp.int32)
  lane = lax.iota(jnp.int32, L)

  off = pl.multiple_of(jnp.minimum(sid * RCH, NR - RCH), 8)
  t_cp = pltpu.make_async_copy(tb.at[pl.ds(off, RCH)],
                               tbn_o.at[pl.ds(off, RCH)], sem_in)
  r_cp = pltpu.make_async_copy(rb.at[pl.ds(off, RCH)],
                               rbn_o.at[pl.ds(off, RCH)], sem_in)

  @pl.when(cid == 0)
  def _():
    t_cp.start()

  @pl.when(cid == 1)
  def _():
    r_cp.start()

  pltpu.sync_copy(idx16, idx_v)
  pltpu.sync_copy(ci, ci_v)
  pltpu.sync_copy(cf, cf_v)
  pltpu.sync_copy(iidx, ii_v)

  @pl.when(sid == 0)
  def _():
    for b_h, sh in ((b1t, shA_t), (b2t, shB_t), (b1r, shA_r), (b2r, shB_r)):
      pltpu.sync_copy(b_h, bias_v)
      pltpu.sync_copy(bias_v, sh)
    for b_h, sh in ((b3t, sh3_t), (b3r, sh3_r)):
      pltpu.sync_copy(b_h, p3_v)
      pltpu.sync_copy(p3_v, sh)

  def bcast(src_ref, base, k, relu):
    v = plsc.load_gather(src_ref, [jnp.full((L,), base + k, jnp.int32)])
    return jnp.maximum(v, 0.0) if relu else v

  def layer_big(src_ref, base, relu, w_ref, sh_acc):
    svs = [bcast(src_ref, base, k, relu) for k in range(L)]

    def jc_body(jc, carry):
      acc = None
      for k in range(L):
        wk = plsc.load_gather(w_ref, [jnp.full((L,), k, jnp.int32),
                                      jc * L + lane])
        acc = svs[k] * wk if acc is None else acc + svs[k] * wk
      part_v[pl.ds(jc * L, L)] = acc
      return carry

    lax.fori_loop(0, L, jc_body, 0, unroll=2)
    pltpu.sync_copy(part_v.at[pl.ds(0, 128)], sh_acc.at[ii_v.at[0]],
                    add=True)
    pltpu.sync_copy(part_v.at[pl.ds(128, 128)], sh_acc.at[ii_v.at[1]],
                    add=True)

  def layer_small(src_ref, base, w_ref, sh_acc):
    svs = [bcast(src_ref, base, k, True) for k in range(L)]
    acc = None
    for k in range(L):
      wk = plsc.load_gather(w_ref, [jnp.full((L,), k, jnp.int32), lane])
      acc = svs[k] * wk if acc is None else acc + svs[k] * wk
    p3_v[...] = acc
    pltpu.sync_copy(p3_v, sh_acc.at[ci_v.at[7]], add=True)

  pltpu.sync_copy(embt.at[idx_v], e_v)
  pltpu.sync_copy(w1t.at[pl.ds(sid * L, L), :], w1_v)
  pltpu.sync_copy(w2t.at[pl.ds(sid * L, L), :], w2_v)
  pltpu.sync_copy(w3t.at[pl.ds(sid * L, L), :], w3_v)
  plsc.subcore_barrier()

  hc_v[pl.ds(L, L)] = plsc.load_gather(e_v, [z16, sid * L + lane])
  layer_big(hc_v, L, False, w1_v, shA_t)
  plsc.subcore_barrier()
  pltpu.sync_copy(shA_t.at[pl.ds(sid * L, L)], hc_v.at[pl.ds(L, L)])
  layer_big(hc_v, L, True, w2_v, shB_t)
  plsc.subcore_barrier()
  pltpu.sync_copy(shB_t.at[pl.ds(sid * L, L)], hc_v.at[pl.ds(L, L)])
  layer_small(hc_v, L, w3_v, sh3_t)

  pltpu.sync_copy(embr.at[idx_v], e_v)
  pltpu.sync_copy(w1r.at[pl.ds(sid * L, L), :], w1_v)
  pltpu.sync_copy(w2r.at[pl.ds(sid * L, L), :], w2_v)
  pltpu.sync_copy(w3r.at[pl.ds(sid * L, L), :], w3_v)

  hc_v[pl.ds(L, L)] = plsc.load_gather(e_v, [z16, sid * L + lane])
  layer_big(hc_v, L, False, w1_v, shA_r)
  plsc.subcore_barrier()
  pltpu.sync_copy(shA_r.at[pl.ds(sid * L, L)], hc_v.at[pl.ds(L, L)])
  layer_big(hc_v, L, True, w2_v, shB_r)
  plsc.subcore_barrier()
  pltpu.sync_copy(shB_r.at[pl.ds(sid * L, L)], hc_v.at[pl.ds(L, L)])
  layer_small(hc_v, L, w3_v, sh3_r)
  plsc.subcore_barrier()

  pltpu.sync_copy(sh3_t, t_v)
  pltpu.sync_copy(sh3_r, r_v)

  rr = r_v[...]
  s = jnp.sum(jnp.where(lane < 4, rr * rr, 0.0))
  sv = jnp.full((L,), s)
  sv_safe = jnp.maximum(sv, 1e-37)
  bits = lax.bitcast_convert_type(sv_safe, jnp.int32)
  y = lax.bitcast_convert_type(
      jnp.full((L,), 0x5F3759DF, jnp.int32) - (bits >> 1), jnp.float32)
  y = y * (1.5 - 0.5 * sv_safe * y * y)
  y = y * (1.5 - 0.5 * sv_safe * y * y)
  y = y * (1.5 - 0.5 * sv_safe * y * y)
  y = y * (1.5 - 0.5 * sv_safe * y * y)
  norm = sv * y
  rq = rr * (1.0 / (norm + 1e-8))
  r_v[...] = rq

  ra = plsc.load_gather(r_v, [ci_v[0]])
  rb_q = plsc.load_gather(r_v, [ci_v[1]])
  rc = plsc.load_gather(r_v, [ci_v[2]])
  rd = plsc.load_gather(r_v, [ci_v[3]])
  tg = plsc.load_gather(t_v, [ci_v[4]])
  bias_v[pl.ds(0, L)] = (cf_v[0] + cf_v[1] * ra * rb_q + cf_v[2] * rc * rd
                         + cf_v[3] * tg)

  @pl.when(cid == 0)
  def _():
    t_cp.wait()

  @pl.when(cid == 1)
  def _():
    r_cp.wait()

  plsc.subcore_barrier()

  idx_s = jnp.max(idx_v[...])
  blk = pl.multiple_of(idx_s - lax.rem(idx_s, 8), 8)
  row_l = idx_v[...] - blk

  @pl.when((cid == 0) & (sid == 0))
  def _():
    pltpu.sync_copy(bias_v.at[pl.ds(0, L)], c2w_o)
    pltpu.sync_copy(tbn_o.at[pl.ds(blk, 8)], bt_v)
    plsc.store_scatter(bt_v, [row_l, lane], t_v[...], mask=lane < 3)
    pltpu.sync_copy(bt_v, tbn_o.at[pl.ds(blk, 8)])

  @pl.when((cid == 1) & (sid == 0))
  def _():
    pltpu.sync_copy(rbn_o.at[pl.ds(blk, 8)], br_v)
    plsc.store_scatter(br_v, [row_l, lane], r_v[...], mask=lane < 4)
    pltpu.sync_copy(br_v, rbn_o.at[pl.ds(blk, 8)])


def kernel(cam_id, emb_t, W1_t, b1_t, W2_t, b2_t, W3_t, b3_t,
           emb_r, W1_r, b1_r, W2_r, b2_r, W3_r, b3_r, t_buf, r_buf):
  n = t_buf.shape[0]
  idx = jnp.asarray(cam_id, jnp.int32) - 1
  idx16 = jnp.full((16,), idx, jnp.int32)
  w3tp = jnp.pad(W3_t, ((0, 0), (0, 16 - W3_t.shape[1])))
  b3tp = jnp.pad(b3_t, (0, 16 - b3_t.shape[0]))
  w3rp = jnp.pad(W3_r, ((0, 0), (0, 16 - W3_r.shape[1])))
  b3rp = jnp.pad(b3_r, (0, 16 - b3_r.shape[0]))
  ci = jnp.asarray(_CI)
  cf = jnp.asarray(_CF)
  iidx = jnp.asarray(_IIDX)

  mesh = plsc.VectorSubcoreMesh(core_axis_name="c", subcore_axis_name="s")
  f = pl.kernel(
      _body,
      out_type=(
          jax.ShapeDtypeStruct((16,), jnp.float32),
          jax.ShapeDtypeStruct((NR, 3), jnp.float32),
          jax.ShapeDtypeStruct((NR, 4), jnp.float32),
      ),
      mesh=mesh,
      compiler_params=pltpu.CompilerParams(needs_layout_passes=False),
      scratch_types=[
          pltpu.VMEM((16,), jnp.int32),
          pltpu.VMEM((8, 16), jnp.int32),
          pltpu.VMEM((4, 16), jnp.float32),
          pltpu.VMEM((2, 128), jnp.int32),
          pltpu.VMEM((16, E), jnp.float32),
          pltpu.VMEM((L, E), jnp.float32),
          pltpu.VMEM((L, E), jnp.float32),
          pltpu.VMEM((L, L), jnp.float32),
          pltpu.VMEM((E,), jnp.float32),
          pltpu.VMEM((L,), jnp.float32),
          pltpu.VMEM((2 * L,), jnp.float32),
          pltpu.VMEM((E,), jnp.float32),
          pltpu.VMEM((L,), jnp.float32),
          pltpu.VMEM((L,), jnp.float32),
          pltpu.VMEM((8, 3), jnp.float32),
          pltpu.VMEM((8, 4), jnp.float32),
          pltpu.VMEM_SHARED((E,), jnp.float32),
          pltpu.VMEM_SHARED((E,), jnp.float32),
          pltpu.VMEM_SHARED((L,), jnp.float32),
          pltpu.VMEM_SHARED((E,), jnp.float32),
          pltpu.VMEM_SHARED((E,), jnp.float32),
          pltpu.VMEM_SHARED((L,), jnp.float32),
          pltpu.SemaphoreType.DMA,
          pltpu.SemaphoreType.DMA,
      ],
  )
  c2w16, tbn, rbn = f(idx16, emb_t, W1_t, b1_t, W2_t, b2_t, w3tp, b3tp,
                      emb_r, W1_r, b1_r, W2_r, b2_r, w3rp, b3rp,
                      t_buf, r_buf, ci, cf, iidx)
  return (c2w16.reshape(4, 4), tbn, rbn)
